# baseline (speedup 1.0000x reference)
import functools

import jax
import jax.numpy as jnp
from jax import lax
from jax.experimental import pallas as pl
from jax.experimental.pallas import tpu as pltpu
from jax.experimental.pallas import tpu_sc as plsc

_TILE = 3584


def _vq_labels_body(x_ref, pal_ref, iota_ref, lab_ref):
    xv = x_ref[0]
    pal = pal_ref[...]
    psq = jnp.sum(pal * pal, axis=1, keepdims=True)
    psq_hi = psq.astype(jnp.bfloat16)
    r1 = psq - psq_hi.astype(jnp.float32)
    psq_mid = r1.astype(jnp.bfloat16)
    psq_lo = (r1 - psq_mid.astype(jnp.float32)).astype(jnp.bfloat16)
    amat = jnp.concatenate(
        [(-2.0 * pal).astype(jnp.bfloat16), psq_hi, psq_mid, psq_lo],
        axis=1)
    T = xv.shape[1]
    ones3 = jnp.ones((3, T), dtype=jnp.bfloat16)
    xa = jnp.concatenate([xv.astype(jnp.bfloat16), ones3], axis=0)
    s = jax.lax.dot_general(
        amat, xa, (((1,), (0,)), ((), ())),
        preferred_element_type=jnp.float32)
    xsq = jnp.sum(xv * xv, axis=0, keepdims=True)
    K = pal.shape[0]
    iota = iota_ref[...]
    m0 = jnp.maximum(jnp.min(s, axis=0, keepdims=True), -xsq)
    labels = jnp.min(jnp.where(s <= m0, iota, float(K)), axis=0)
    lab_ref[0] = labels[None, :].astype(jnp.int32)


def _labels_tc(xr, palette):
    B, C, HW = xr.shape
    grid = (B, HW // _TILE)
    iota_col = jnp.arange(palette.shape[0], dtype=jnp.float32)[:, None]
    return pl.pallas_call(
        _vq_labels_body,
        grid=grid,
        in_specs=[
            pl.BlockSpec((1, C, _TILE), lambda b, i: (b, 0, i)),
            pl.BlockSpec(palette.shape, lambda b, i: (0, 0)),
            pl.BlockSpec(iota_col.shape, lambda b, i: (0, 0)),
        ],
        out_specs=pl.BlockSpec((1, 1, _TILE), lambda b, i: (b, 0, i)),
        out_shape=jax.ShapeDtypeStruct((B, 1, HW), jnp.int32),
    )(xr, palette, iota_col)


_NC = 2
_NS = 16
_NW = _NC * _NS


def _sc_gather_body(chunk, hw, lab_hbm, pal_hbm, out_hbm, pal_v, lab_v,
                    out_v0, out_v1, out_v2):
    out_v = (out_v0, out_v1, out_v2)
    wid = lax.axis_index("c") * _NS + lax.axis_index("s")
    wpb = hw // chunk
    b = wid // wpb
    off = (wid % wpb) * chunk
    pix = b * hw + off
    pltpu.sync_copy(pal_hbm, pal_v)
    pltpu.sync_copy(lab_hbm.at[pl.ds(pix, chunk)], lab_v)

    def body(i, carry):
        l16 = lab_v[pl.ds(i * 16, 16)]
        base = l16 * 3
        for ch in range(3):
            out_v[ch][pl.ds(i * 16, 16)] = plsc.load_gather(
                pal_v, [base + ch])
        return carry

    lax.fori_loop(0, chunk // 16, body, 0)
    for ch in range(3):
        pltpu.sync_copy(out_v[ch],
                        out_hbm.at[pl.ds((b * 3 + ch) * hw + off, chunk)])


def _gather_sc(labels, palette, hw):
    n = labels.shape[0]
    chunk = n // _NW
    mesh = plsc.VectorSubcoreMesh(core_axis_name="c", subcore_axis_name="s")
    fn = functools.partial(
        pl.kernel,
        mesh=mesh,
        compiler_params=pltpu.CompilerParams(needs_layout_passes=False),
        out_type=jax.ShapeDtypeStruct((3 * n,), jnp.float32),
        scratch_types=[
            pltpu.VMEM((palette.shape[0] * palette.shape[1],), jnp.float32),
            pltpu.VMEM((chunk,), jnp.int32),
            pltpu.VMEM((chunk,), jnp.float32),
            pltpu.VMEM((chunk,), jnp.float32),
            pltpu.VMEM((chunk,), jnp.float32),
        ],
    )(functools.partial(_sc_gather_body, chunk, hw))
    return fn(labels, palette.reshape(-1))


def kernel(x, palette):
    B, C, H, W = x.shape
    HW = H * W
    xr = x.reshape(B, C, HW)
    half = B // 2
    outs = []
    for p in range(2):
        xp = xr[p * half:(p + 1) * half]
        labels = _labels_tc(xp, palette).reshape(half * HW)
        outs.append(_gather_sc(labels, palette, HW))
    out = jnp.concatenate(outs)
    return out.reshape(B, C, H, W)


# speedup vs baseline: 1.0963x; 1.0963x over previous
import functools

import jax
import jax.numpy as jnp
from jax import lax
from jax.experimental import pallas as pl
from jax.experimental.pallas import tpu as pltpu
from jax.experimental.pallas import tpu_sc as plsc

_TILE = 7168


def _vq_labels_body(x_ref, pal_ref, iota_ref, lab_ref):
    xv = x_ref[0]
    pal = pal_ref[...]
    psq = jnp.sum(pal * pal, axis=1, keepdims=True)
    psq_hi = psq.astype(jnp.bfloat16)
    r1 = psq - psq_hi.astype(jnp.float32)
    psq_mid = r1.astype(jnp.bfloat16)
    psq_lo = (r1 - psq_mid.astype(jnp.float32)).astype(jnp.bfloat16)
    amat = jnp.concatenate(
        [(-2.0 * pal).astype(jnp.bfloat16), psq_hi, psq_mid, psq_lo],
        axis=1)
    T = xv.shape[1]
    ones3 = jnp.ones((3, T), dtype=jnp.bfloat16)
    xa = jnp.concatenate([xv.astype(jnp.bfloat16), ones3], axis=0)
    s = jax.lax.dot_general(
        amat, xa, (((1,), (0,)), ((), ())),
        preferred_element_type=jnp.float32)
    xsq = jnp.sum(xv * xv, axis=0, keepdims=True)
    K = pal.shape[0]
    iota = iota_ref[...]
    m0 = jnp.maximum(jnp.min(s, axis=0, keepdims=True), -xsq)
    labels = jnp.min(jnp.where(s <= m0, iota, float(K)), axis=0)
    lab_ref[0] = labels[None, :].astype(jnp.int32)


def _labels_tc(xr, palette):
    B, C, HW = xr.shape
    grid = (B, HW // _TILE)
    iota_col = jnp.arange(palette.shape[0], dtype=jnp.float32)[:, None]
    return pl.pallas_call(
        _vq_labels_body,
        grid=grid,
        in_specs=[
            pl.BlockSpec((1, C, _TILE), lambda b, i: (b, 0, i)),
            pl.BlockSpec(palette.shape, lambda b, i: (0, 0)),
            pl.BlockSpec(iota_col.shape, lambda b, i: (0, 0)),
        ],
        out_specs=pl.BlockSpec((1, 1, _TILE), lambda b, i: (b, 0, i)),
        out_shape=jax.ShapeDtypeStruct((B, 1, HW), jnp.int32),
    )(xr, palette, iota_col)


_NC = 2
_NS = 16
_NW = _NC * _NS


def _sc_gather_body(chunk, hw, lab_hbm, pal_hbm, out_hbm, pal_v, lab_v,
                    out_v0, out_v1, out_v2):
    out_v = (out_v0, out_v1, out_v2)
    wid = lax.axis_index("c") * _NS + lax.axis_index("s")
    wpb = hw // chunk
    b = wid // wpb
    off = (wid % wpb) * chunk
    pix = b * hw + off
    pltpu.sync_copy(pal_hbm, pal_v)
    pltpu.sync_copy(lab_hbm.at[pl.ds(pix, chunk)], lab_v)

    def body(i, carry):
        l16 = lab_v[pl.ds(i * 16, 16)]
        base = l16 * 3
        for ch in range(3):
            out_v[ch][pl.ds(i * 16, 16)] = plsc.load_gather(
                pal_v, [base + ch])
        return carry

    lax.fori_loop(0, chunk // 16, body, 0)
    for ch in range(3):
        pltpu.sync_copy(out_v[ch],
                        out_hbm.at[pl.ds((b * 3 + ch) * hw + off, chunk)])


def _gather_sc(labels, palette, hw):
    n = labels.shape[0]
    chunk = n // _NW
    mesh = plsc.VectorSubcoreMesh(core_axis_name="c", subcore_axis_name="s")
    fn = functools.partial(
        pl.kernel,
        mesh=mesh,
        compiler_params=pltpu.CompilerParams(needs_layout_passes=False),
        out_type=jax.ShapeDtypeStruct((3 * n,), jnp.float32),
        scratch_types=[
            pltpu.VMEM((palette.shape[0] * palette.shape[1],), jnp.float32),
            pltpu.VMEM((chunk,), jnp.int32),
            pltpu.VMEM((chunk,), jnp.float32),
            pltpu.VMEM((chunk,), jnp.float32),
            pltpu.VMEM((chunk,), jnp.float32),
        ],
    )(functools.partial(_sc_gather_body, chunk, hw))
    return fn(labels, palette.reshape(-1))


def kernel(x, palette):
    B, C, H, W = x.shape
    HW = H * W
    xr = x.reshape(B, C, HW)
    labels = _labels_tc(xr, palette).reshape(B * HW)
    out = _gather_sc(labels, palette, HW)
    return out.reshape(B, C, H, W)


# speedup vs baseline: 1.1205x; 1.0221x over previous
import functools

import jax
import jax.numpy as jnp
from jax import lax
from jax.experimental import pallas as pl
from jax.experimental.pallas import tpu as pltpu
from jax.experimental.pallas import tpu_sc as plsc

_TILE = 12544


def _vq_labels_body(x_ref, pal_ref, iota_ref, lab_ref):
    xv = x_ref[0]
    pal = pal_ref[...]
    psq = jnp.sum(pal * pal, axis=1, keepdims=True)
    psq_hi = psq.astype(jnp.bfloat16)
    r1 = psq - psq_hi.astype(jnp.float32)
    psq_mid = r1.astype(jnp.bfloat16)
    psq_lo = (r1 - psq_mid.astype(jnp.float32)).astype(jnp.bfloat16)
    amat = jnp.concatenate(
        [(-2.0 * pal).astype(jnp.bfloat16), psq_hi, psq_mid, psq_lo],
        axis=1)
    T = xv.shape[1]
    ones3 = jnp.ones((3, T), dtype=jnp.bfloat16)
    xa = jnp.concatenate([xv.astype(jnp.bfloat16), ones3], axis=0)
    s = jax.lax.dot_general(
        amat, xa, (((1,), (0,)), ((), ())),
        preferred_element_type=jnp.float32)
    xsq = jnp.sum(xv * xv, axis=0, keepdims=True)
    K = pal.shape[0]
    iota = iota_ref[...]
    m0 = jnp.maximum(jnp.min(s, axis=0, keepdims=True), -xsq)
    labels = jnp.min(jnp.where(s <= m0, iota, float(K)), axis=0)
    lab_ref[0] = labels[None, :].astype(jnp.int32)


def _labels_tc(xr, palette):
    B, C, HW = xr.shape
    grid = (B, HW // _TILE)
    iota_col = jnp.arange(palette.shape[0], dtype=jnp.float32)[:, None]
    return pl.pallas_call(
        _vq_labels_body,
        grid=grid,
        in_specs=[
            pl.BlockSpec((1, C, _TILE), lambda b, i: (b, 0, i)),
            pl.BlockSpec(palette.shape, lambda b, i: (0, 0)),
            pl.BlockSpec(iota_col.shape, lambda b, i: (0, 0)),
        ],
        out_specs=pl.BlockSpec((1, 1, _TILE), lambda b, i: (b, 0, i)),
        out_shape=jax.ShapeDtypeStruct((B, 1, HW), jnp.int32),
    )(xr, palette, iota_col)


_NC = 2
_NS = 16
_NW = _NC * _NS


def _sc_gather_body(chunk, hw, lab_hbm, pal_hbm, out_hbm, pal_v, lab_v,
                    out_v0, out_v1, out_v2):
    out_v = (out_v0, out_v1, out_v2)
    wid = lax.axis_index("c") * _NS + lax.axis_index("s")
    wpb = hw // chunk
    b = wid // wpb
    off = (wid % wpb) * chunk
    pix = b * hw + off
    pltpu.sync_copy(pal_hbm, pal_v)
    pltpu.sync_copy(lab_hbm.at[pl.ds(pix, chunk)], lab_v)

    def body(i, carry):
        l16 = lab_v[pl.ds(i * 16, 16)]
        base = l16 * 3
        for ch in range(3):
            out_v[ch][pl.ds(i * 16, 16)] = plsc.load_gather(
                pal_v, [base + ch])
        return carry

    lax.fori_loop(0, chunk // 16, body, 0)
    for ch in range(3):
        pltpu.sync_copy(out_v[ch],
                        out_hbm.at[pl.ds((b * 3 + ch) * hw + off, chunk)])


def _gather_sc(labels, palette, hw):
    n = labels.shape[0]
    chunk = n // _NW
    mesh = plsc.VectorSubcoreMesh(core_axis_name="c", subcore_axis_name="s")
    fn = functools.partial(
        pl.kernel,
        mesh=mesh,
        compiler_params=pltpu.CompilerParams(needs_layout_passes=False),
        out_type=jax.ShapeDtypeStruct((3 * n,), jnp.float32),
        scratch_types=[
            pltpu.VMEM((palette.shape[0] * palette.shape[1],), jnp.float32),
            pltpu.VMEM((chunk,), jnp.int32),
            pltpu.VMEM((chunk,), jnp.float32),
            pltpu.VMEM((chunk,), jnp.float32),
            pltpu.VMEM((chunk,), jnp.float32),
        ],
    )(functools.partial(_sc_gather_body, chunk, hw))
    return fn(labels, palette.reshape(-1))


def kernel(x, palette):
    B, C, H, W = x.shape
    HW = H * W
    xr = x.reshape(B, C, HW)
    labels = _labels_tc(xr, palette).reshape(B * HW)
    out = _gather_sc(labels, palette, HW)
    return out.reshape(B, C, H, W)
